# baseline (speedup 1.0000x reference)
import functools

import jax
import jax.numpy as jnp
from jax import lax
from jax.experimental import pallas as pl
from jax.experimental.pallas import tpu as pltpu
from jax.experimental.pallas import tpu_sc as plsc

N = 10000
E = 320000
D = 128

NC = 2
NS = 16
NW = NC * NS

PADN = 10240
SLC = PADN // NS

CHD = 125
CPTD = E // (NW * CHD)

CH = 125
CPT = E // (NW * CH)
SB = 16
NSB = CPT // SB
NBUF = 2

_mesh = plsc.VectorSubcoreMesh(
    core_axis_name="c", subcore_axis_name="s", num_cores=NC, num_subcores=NS
)


@functools.partial(
    pl.kernel,
    out_type=jax.ShapeDtypeStruct((NC * 2 * PADN,), jnp.float32),
    mesh=_mesh,
    scratch_types=[
        pltpu.VMEM_SHARED((PADN,), jnp.float32),
        pltpu.VMEM_SHARED((PADN,), jnp.float32),
        pltpu.VMEM((CPTD, CHD), jnp.int32),
        pltpu.VMEM((CPTD, CHD), jnp.int32),
        pltpu.VMEM((128,), jnp.float32),
    ],
)
def _deg_kernel(src_hbm, dst_hbm, zeros_hbm, out_hbm, od_sh, id_sh, si_v, di_v, ones_v):
    c = lax.axis_index("c")
    s = lax.axis_index("s")
    w = c * NS + s
    pltpu.sync_copy(zeros_hbm, od_sh.at[pl.ds(s * SLC, SLC)])
    pltpu.sync_copy(zeros_hbm, id_sh.at[pl.ds(s * SLC, SLC)])
    for i in range(8):
        ones_v[pl.ds(i * 16, 16)] = jnp.full((16,), 1.0, jnp.float32)
    pltpu.sync_copy(src_hbm.at[pl.ds(w * CPTD, CPTD)], si_v)
    pltpu.sync_copy(dst_hbm.at[pl.ds(w * CPTD, CPTD)], di_v)
    plsc.subcore_barrier()

    def body(j, carry):
        pltpu.sync_copy(ones_v.at[pl.ds(0, CHD)], od_sh.at[si_v.at[j]], add=True)
        pltpu.sync_copy(ones_v.at[pl.ds(0, CHD)], id_sh.at[di_v.at[j]], add=True)
        return carry

    lax.fori_loop(0, CPTD, body, 0)
    plsc.subcore_barrier()
    base = c * 2 * PADN
    pltpu.sync_copy(od_sh.at[pl.ds(s * SLC, SLC)], out_hbm.at[pl.ds(base + s * SLC, SLC)])
    pltpu.sync_copy(
        id_sh.at[pl.ds(s * SLC, SLC)], out_hbm.at[pl.ds(base + PADN + s * SLC, SLC)]
    )


@functools.partial(
    pl.kernel,
    out_type=jax.ShapeDtypeStruct((NC, PADN, D), jnp.float32),
    mesh=_mesh,
    scratch_types=[
        pltpu.VMEM_SHARED((PADN, D), jnp.float32),
        pltpu.VMEM((NBUF, SB, CH), jnp.int32),
        pltpu.VMEM((NBUF, SB, CH), jnp.int32),
        pltpu.VMEM((NBUF, CH, D), jnp.float32),
        pltpu.SemaphoreType.DMA,
        pltpu.SemaphoreType.DMA,
        pltpu.SemaphoreType.DMA,
        pltpu.SemaphoreType.DMA,
        pltpu.SemaphoreType.DMA,
        pltpu.SemaphoreType.DMA,
    ],
)
def _agg_kernel(h_hbm, src_hbm, dst_hbm, zeros_hbm, out_hbm, agg_sh, si_v, di_v, rows_v, *sems):
    isem = sems[:2]
    gsem = sems[2:4]
    ssem = sems[4:]
    c = lax.axis_index("c")
    s = lax.axis_index("s")
    w = c * NS + s
    row0 = w * CPT
    pltpu.sync_copy(zeros_hbm, agg_sh.at[pl.ds(s * SLC, SLC)])
    pltpu.async_copy(src_hbm.at[pl.ds(row0, SB)], si_v.at[0], isem[0])
    pltpu.async_copy(dst_hbm.at[pl.ds(row0, SB)], di_v.at[0], isem[0])
    plsc.subcore_barrier()

    for sb in range(NSB):
        p = sb % 2
        blk = row0 + sb * SB
        pltpu.make_async_copy(src_hbm.at[pl.ds(blk, SB)], si_v.at[p], isem[p]).wait()
        pltpu.make_async_copy(dst_hbm.at[pl.ds(blk, SB)], di_v.at[p], isem[p]).wait()
        if sb + 1 < NSB:
            nblk = blk + SB
            pltpu.async_copy(src_hbm.at[pl.ds(nblk, SB)], si_v.at[1 - p], isem[1 - p])
            pltpu.async_copy(dst_hbm.at[pl.ds(nblk, SB)], di_v.at[1 - p], isem[1 - p])

        pltpu.async_copy(h_hbm.at[si_v.at[p, 0]], rows_v.at[0], gsem[0])
        pltpu.async_copy(h_hbm.at[si_v.at[p, 1]], rows_v.at[1], gsem[1])
        for t in range(SB // 2):
            scat = []
            for b in range(2):
                jj = 2 * t + b
                pltpu.make_async_copy(h_hbm.at[si_v.at[p, jj]], rows_v.at[b], gsem[b]).wait()
                scat.append(
                    pltpu.async_copy(rows_v.at[b], agg_sh.at[di_v.at[p, jj]], ssem[b], add=True)
                )
            for b in range(2):
                jj = 2 * t + b
                scat[b].wait()
                if jj + 2 < SB:
                    pltpu.async_copy(h_hbm.at[si_v.at[p, jj + 2]], rows_v.at[b], gsem[b])
    plsc.subcore_barrier()
    pltpu.sync_copy(
        agg_sh.at[pl.ds(s * SLC, SLC)],
        out_hbm.at[c, pl.ds(s * SLC, SLC)],
    )


BR = 1000


def _prep_body(x_ref, odp_ref, o_ref):
    od = odp_ref[0] + odp_ref[1]
    o_ref[...] = x_ref[...] * lax.rsqrt(jnp.maximum(od, 1.0))


_prep = pl.pallas_call(
    _prep_body,
    grid=(N // BR,),
    in_specs=[
        pl.BlockSpec((BR, D), lambda i: (i, 0)),
        pl.BlockSpec((2, BR, 1), lambda i: (0, i, 0)),
    ],
    out_specs=pl.BlockSpec((BR, D), lambda i: (i, 0)),
    out_shape=jax.ShapeDtypeStruct((N, D), jnp.float32),
)


def _make_dense(with_out_scale):
    def body(aggp_ref, idp_ref, odp_ref, w_ref, b_ref, o_ref):
        agg = aggp_ref[0] + aggp_ref[1]
        ideg = idp_ref[0] + idp_ref[1]
        x = agg * lax.rsqrt(jnp.maximum(ideg, 1.0))
        y = jnp.dot(x, w_ref[...], preferred_element_type=jnp.float32) + b_ref[...]
        y = jnp.maximum(y, 0.0)
        if with_out_scale:
            od = odp_ref[0] + odp_ref[1]
            y = y * lax.rsqrt(jnp.maximum(od, 1.0))
        o_ref[...] = y

    return pl.pallas_call(
        body,
        grid=(N // BR,),
        in_specs=[
            pl.BlockSpec((2, BR, D), lambda i: (0, i, 0)),
            pl.BlockSpec((2, BR, 1), lambda i: (0, i, 0)),
            pl.BlockSpec((2, BR, 1), lambda i: (0, i, 0)),
            pl.BlockSpec((D, D), lambda i: (0, 0)),
            pl.BlockSpec((1, D), lambda i: (0, 0)),
        ],
        out_specs=pl.BlockSpec((BR, D), lambda i: (i, 0)),
        out_shape=jax.ShapeDtypeStruct((N, D), jnp.float32),
    )


_dense_scaled = _make_dense(True)
_dense_plain = _make_dense(False)


@jax.jit
def kernel(inputs, edge_index, W1, b1, W2, b2):
    src = edge_index[0].astype(jnp.int32)
    dst = edge_index[1].astype(jnp.int32)
    src_a = src.reshape(E // CH, CH)
    dst_a = dst.reshape(E // CH, CH)
    zeros_deg = jnp.zeros((SLC,), jnp.float32)
    zeros_agg = jnp.zeros((SLC, D), jnp.float32)

    deg = _deg_kernel(src_a, dst_a, zeros_deg).reshape(NC, 2, PADN)
    odp = deg[:, 0, :N].reshape(NC, N, 1)
    idp = deg[:, 1, :N].reshape(NC, N, 1)
    b1r = b1.reshape(1, D)
    b2r = b2.reshape(1, D)

    h1n = _prep(inputs, odp)
    agg1 = _agg_kernel(h1n, src_a, dst_a, zeros_agg)
    h2n = _dense_scaled(agg1, idp, odp, W1, b1r)
    agg2 = _agg_kernel(h2n, src_a, dst_a, zeros_agg)
    return _dense_plain(agg2, idp, odp, W2, b2r)


# speedup vs baseline: 1.2182x; 1.2182x over previous
import functools

import jax
import jax.numpy as jnp
from jax import lax
from jax.experimental import pallas as pl
from jax.experimental.pallas import tpu as pltpu
from jax.experimental.pallas import tpu_sc as plsc

N = 10000
E = 320000
D = 128

NC = 2
NS = 16
NW = NC * NS

PADN = 10240
SLC = PADN // NS

CHD = 125
CPTD = E // (NW * CHD)

CH = 125
CPT = E // (NW * CH)
SB = 16
NSB = CPT // SB
NBUF = 2

_mesh = plsc.VectorSubcoreMesh(
    core_axis_name="c", subcore_axis_name="s", num_cores=NC, num_subcores=NS
)


@functools.partial(
    pl.kernel,
    out_type=jax.ShapeDtypeStruct((NC * 2 * PADN,), jnp.float32),
    mesh=_mesh,
    scratch_types=[
        pltpu.VMEM_SHARED((PADN,), jnp.float32),
        pltpu.VMEM_SHARED((PADN,), jnp.float32),
        pltpu.VMEM((CPTD, CHD), jnp.int32),
        pltpu.VMEM((CPTD, CHD), jnp.int32),
        pltpu.VMEM((128,), jnp.float32),
        pltpu.SemaphoreType.DMA,
    ],
)
def _deg_kernel(src_hbm, dst_hbm, zeros_hbm, out_hbm, od_sh, id_sh, si_v, di_v, ones_v, dsem):
    c = lax.axis_index("c")
    s = lax.axis_index("s")
    w = c * NS + s
    pltpu.sync_copy(zeros_hbm, od_sh.at[pl.ds(s * SLC, SLC)])
    pltpu.sync_copy(zeros_hbm, id_sh.at[pl.ds(s * SLC, SLC)])
    for i in range(8):
        ones_v[pl.ds(i * 16, 16)] = jnp.full((16,), 1.0, jnp.float32)
    pltpu.sync_copy(src_hbm.at[pl.ds(w * CPTD, CPTD)], si_v)
    pltpu.sync_copy(dst_hbm.at[pl.ds(w * CPTD, CPTD)], di_v)
    plsc.subcore_barrier()

    def body(j, carry):
        pltpu.async_copy(ones_v.at[pl.ds(0, CHD)], od_sh.at[si_v.at[j]], dsem, add=True)
        pltpu.async_copy(ones_v.at[pl.ds(0, CHD)], id_sh.at[di_v.at[j]], dsem, add=True)
        return carry

    lax.fori_loop(0, CPTD, body, 0)

    def drain(j, carry):
        pltpu.make_async_copy(ones_v.at[pl.ds(0, CHD)], od_sh.at[si_v.at[j]], dsem).wait()
        pltpu.make_async_copy(ones_v.at[pl.ds(0, CHD)], id_sh.at[di_v.at[j]], dsem).wait()
        return carry

    lax.fori_loop(0, CPTD, drain, 0)
    plsc.subcore_barrier()
    base = c * 2 * PADN
    pltpu.sync_copy(od_sh.at[pl.ds(s * SLC, SLC)], out_hbm.at[pl.ds(base + s * SLC, SLC)])
    pltpu.sync_copy(
        id_sh.at[pl.ds(s * SLC, SLC)], out_hbm.at[pl.ds(base + PADN + s * SLC, SLC)]
    )


@functools.partial(
    pl.kernel,
    out_type=jax.ShapeDtypeStruct((NC, PADN, D), jnp.float32),
    mesh=_mesh,
    scratch_types=[
        pltpu.VMEM_SHARED((PADN, D), jnp.float32),
        pltpu.VMEM((NBUF, SB, CH), jnp.int32),
        pltpu.VMEM((NBUF, SB, CH), jnp.int32),
        pltpu.VMEM((NBUF, CH, D), jnp.float32),
        pltpu.SemaphoreType.DMA,
        pltpu.SemaphoreType.DMA,
        pltpu.SemaphoreType.DMA,
        pltpu.SemaphoreType.DMA,
        pltpu.SemaphoreType.DMA,
        pltpu.SemaphoreType.DMA,
    ],
)
def _agg_kernel(h_hbm, src_hbm, dst_hbm, zeros_hbm, out_hbm, agg_sh, si_v, di_v, rows_v, *sems):
    isem = sems[:2]
    gsem = sems[2:4]
    ssem = sems[4:]
    c = lax.axis_index("c")
    s = lax.axis_index("s")
    w = c * NS + s
    row0 = w * CPT
    pltpu.sync_copy(zeros_hbm, agg_sh.at[pl.ds(s * SLC, SLC)])
    pltpu.async_copy(src_hbm.at[pl.ds(row0, SB)], si_v.at[0], isem[0])
    pltpu.async_copy(dst_hbm.at[pl.ds(row0, SB)], di_v.at[0], isem[0])
    plsc.subcore_barrier()

    for sb in range(NSB):
        p = sb % 2
        blk = row0 + sb * SB
        pltpu.make_async_copy(src_hbm.at[pl.ds(blk, SB)], si_v.at[p], isem[p]).wait()
        pltpu.make_async_copy(dst_hbm.at[pl.ds(blk, SB)], di_v.at[p], isem[p]).wait()
        if sb + 1 < NSB:
            nblk = blk + SB
            pltpu.async_copy(src_hbm.at[pl.ds(nblk, SB)], si_v.at[1 - p], isem[1 - p])
            pltpu.async_copy(dst_hbm.at[pl.ds(nblk, SB)], di_v.at[1 - p], isem[1 - p])

        pltpu.async_copy(h_hbm.at[si_v.at[p, 0]], rows_v.at[0], gsem[0])
        pltpu.async_copy(h_hbm.at[si_v.at[p, 1]], rows_v.at[1], gsem[1])
        for jj in range(SB):
            b = jj % 2
            pltpu.make_async_copy(h_hbm.at[si_v.at[p, jj]], rows_v.at[b], gsem[b]).wait()
            pltpu.sync_copy(rows_v.at[b], agg_sh.at[di_v.at[p, jj]], add=True)
            if jj + 2 < SB:
                pltpu.async_copy(h_hbm.at[si_v.at[p, jj + 2]], rows_v.at[b], gsem[b])
    plsc.subcore_barrier()
    pltpu.sync_copy(
        agg_sh.at[pl.ds(s * SLC, SLC)],
        out_hbm.at[c, pl.ds(s * SLC, SLC)],
    )


BR = 1000


def _prep_body(x_ref, odp_ref, o_ref):
    od = odp_ref[0] + odp_ref[1]
    o_ref[...] = x_ref[...] * lax.rsqrt(jnp.maximum(od, 1.0))


_prep = pl.pallas_call(
    _prep_body,
    grid=(N // BR,),
    in_specs=[
        pl.BlockSpec((BR, D), lambda i: (i, 0)),
        pl.BlockSpec((2, BR, 1), lambda i: (0, i, 0)),
    ],
    out_specs=pl.BlockSpec((BR, D), lambda i: (i, 0)),
    out_shape=jax.ShapeDtypeStruct((N, D), jnp.float32),
)


def _make_dense(with_out_scale):
    def body(aggp_ref, idp_ref, odp_ref, w_ref, b_ref, o_ref):
        agg = aggp_ref[0] + aggp_ref[1]
        ideg = idp_ref[0] + idp_ref[1]
        x = agg * lax.rsqrt(jnp.maximum(ideg, 1.0))
        y = jnp.dot(x, w_ref[...], preferred_element_type=jnp.float32) + b_ref[...]
        y = jnp.maximum(y, 0.0)
        if with_out_scale:
            od = odp_ref[0] + odp_ref[1]
            y = y * lax.rsqrt(jnp.maximum(od, 1.0))
        o_ref[...] = y

    return pl.pallas_call(
        body,
        grid=(N // BR,),
        in_specs=[
            pl.BlockSpec((2, BR, D), lambda i: (0, i, 0)),
            pl.BlockSpec((2, BR, 1), lambda i: (0, i, 0)),
            pl.BlockSpec((2, BR, 1), lambda i: (0, i, 0)),
            pl.BlockSpec((D, D), lambda i: (0, 0)),
            pl.BlockSpec((1, D), lambda i: (0, 0)),
        ],
        out_specs=pl.BlockSpec((BR, D), lambda i: (i, 0)),
        out_shape=jax.ShapeDtypeStruct((N, D), jnp.float32),
    )


_dense_scaled = _make_dense(True)
_dense_plain = _make_dense(False)


@jax.jit
def kernel(inputs, edge_index, W1, b1, W2, b2):
    src = edge_index[0].astype(jnp.int32)
    dst = edge_index[1].astype(jnp.int32)
    src_a = src.reshape(E // CH, CH)
    dst_a = dst.reshape(E // CH, CH)
    zeros_deg = jnp.zeros((SLC,), jnp.float32)
    zeros_agg = jnp.zeros((SLC, D), jnp.float32)

    deg = _deg_kernel(src_a, dst_a, zeros_deg).reshape(NC, 2, PADN)
    odp = deg[:, 0, :N].reshape(NC, N, 1)
    idp = deg[:, 1, :N].reshape(NC, N, 1)
    b1r = b1.reshape(1, D)
    b2r = b2.reshape(1, D)

    h1n = _prep(inputs, odp)
    agg1 = _agg_kernel(h1n, src_a, dst_a, zeros_agg)
    h2n = _dense_scaled(agg1, idp, odp, W1, b1r)
    agg2 = _agg_kernel(h2n, src_a, dst_a, zeros_agg)
    return _dense_plain(agg2, idp, odp, W2, b2r)


# speedup vs baseline: 1.2605x; 1.0348x over previous
import functools

import jax
import jax.numpy as jnp
from jax import lax
from jax.experimental import pallas as pl
from jax.experimental.pallas import tpu as pltpu
from jax.experimental.pallas import tpu_sc as plsc

N = 10000
E = 320000
D = 128

NC = 2
NS = 16
NW = NC * NS

PADN = 10240
SLC = PADN // NS

CHD = 125
CPTD = E // (NW * CHD)

CH = 125
CPT = E // (NW * CH)
SB = 16
NSB = CPT // SB
NBUF = 2

_mesh = plsc.VectorSubcoreMesh(
    core_axis_name="c", subcore_axis_name="s", num_cores=NC, num_subcores=NS
)


@functools.partial(
    pl.kernel,
    out_type=jax.ShapeDtypeStruct((NC * 2 * PADN,), jnp.float32),
    mesh=_mesh,
    scratch_types=[
        pltpu.VMEM_SHARED((PADN,), jnp.float32),
        pltpu.VMEM_SHARED((PADN,), jnp.float32),
        pltpu.VMEM((CPTD, CHD), jnp.int32),
        pltpu.VMEM((CPTD, CHD), jnp.int32),
        pltpu.VMEM((128,), jnp.float32),
        pltpu.SemaphoreType.DMA,
    ],
)
def _deg_kernel(src_hbm, dst_hbm, zeros_hbm, out_hbm, od_sh, id_sh, si_v, di_v, ones_v, dsem):
    c = lax.axis_index("c")
    s = lax.axis_index("s")
    w = c * NS + s
    pltpu.sync_copy(zeros_hbm, od_sh.at[pl.ds(s * SLC, SLC)])
    pltpu.sync_copy(zeros_hbm, id_sh.at[pl.ds(s * SLC, SLC)])
    for i in range(8):
        ones_v[pl.ds(i * 16, 16)] = jnp.full((16,), 1.0, jnp.float32)
    pltpu.sync_copy(src_hbm.at[pl.ds(w * CPTD, CPTD)], si_v)
    pltpu.sync_copy(dst_hbm.at[pl.ds(w * CPTD, CPTD)], di_v)
    plsc.subcore_barrier()

    def body(j, carry):
        pltpu.async_copy(ones_v.at[pl.ds(0, CHD)], od_sh.at[si_v.at[j]], dsem, add=True)
        pltpu.async_copy(ones_v.at[pl.ds(0, CHD)], id_sh.at[di_v.at[j]], dsem, add=True)
        return carry

    lax.fori_loop(0, CPTD, body, 0)

    def drain(j, carry):
        pltpu.make_async_copy(ones_v.at[pl.ds(0, CHD)], od_sh.at[si_v.at[j]], dsem).wait()
        pltpu.make_async_copy(ones_v.at[pl.ds(0, CHD)], id_sh.at[di_v.at[j]], dsem).wait()
        return carry

    lax.fori_loop(0, CPTD, drain, 0)
    plsc.subcore_barrier()
    base = c * 2 * PADN
    pltpu.sync_copy(od_sh.at[pl.ds(s * SLC, SLC)], out_hbm.at[pl.ds(base + s * SLC, SLC)])
    pltpu.sync_copy(
        id_sh.at[pl.ds(s * SLC, SLC)], out_hbm.at[pl.ds(base + PADN + s * SLC, SLC)]
    )


@functools.partial(
    pl.kernel,
    out_type=jax.ShapeDtypeStruct((NC, PADN, D), jnp.float32),
    mesh=_mesh,
    scratch_types=[
        pltpu.VMEM_SHARED((PADN, D), jnp.float32),
        pltpu.VMEM((3, SB, CH), jnp.int32),
        pltpu.VMEM((3, SB, CH), jnp.int32),
        pltpu.VMEM((NBUF, CH, D), jnp.float32),
        pltpu.SemaphoreType.DMA,
        pltpu.SemaphoreType.DMA,
        pltpu.SemaphoreType.DMA,
        pltpu.SemaphoreType.DMA,
        pltpu.SemaphoreType.DMA,
    ],
)
def _agg_kernel(h_hbm, src_hbm, dst_hbm, zeros_hbm, out_hbm, agg_sh, si_v, di_v, rows_v, *sems):
    isem = sems[:3]
    gsem = sems[3:]
    c = lax.axis_index("c")
    s = lax.axis_index("s")
    w = c * NS + s
    row0 = w * CPT

    def pref(sb):
        pltpu.async_copy(src_hbm.at[pl.ds(row0 + sb * SB, SB)], si_v.at[sb % 3], isem[sb % 3])
        pltpu.async_copy(dst_hbm.at[pl.ds(row0 + sb * SB, SB)], di_v.at[sb % 3], isem[sb % 3])

    def wait_pref(sb):
        pltpu.make_async_copy(
            src_hbm.at[pl.ds(row0 + sb * SB, SB)], si_v.at[sb % 3], isem[sb % 3]
        ).wait()
        pltpu.make_async_copy(
            dst_hbm.at[pl.ds(row0 + sb * SB, SB)], di_v.at[sb % 3], isem[sb % 3]
        ).wait()

    pref(0)
    pltpu.sync_copy(zeros_hbm, agg_sh.at[pl.ds(s * SLC, SLC)])
    plsc.subcore_barrier()

    wait_pref(0)
    pref(1)
    pltpu.async_copy(h_hbm.at[si_v.at[0, 0]], rows_v.at[0], gsem[0])
    pltpu.async_copy(h_hbm.at[si_v.at[0, 1]], rows_v.at[1], gsem[1])
    for g in range(CPT):
        b = g % 2
        sb, jj = divmod(g, SB)
        pltpu.make_async_copy(h_hbm.at[si_v.at[sb % 3, jj]], rows_v.at[b], gsem[b]).wait()
        pltpu.sync_copy(rows_v.at[b], agg_sh.at[di_v.at[sb % 3, jj]], add=True)
        g2 = g + 2
        if g2 < CPT:
            sb2, jj2 = divmod(g2, SB)
            if jj2 == 0:
                wait_pref(sb2)
                if sb2 + 1 < NSB:
                    pref(sb2 + 1)
            pltpu.async_copy(h_hbm.at[si_v.at[sb2 % 3, jj2]], rows_v.at[b], gsem[b])
    plsc.subcore_barrier()
    pltpu.sync_copy(
        agg_sh.at[pl.ds(s * SLC, SLC)],
        out_hbm.at[c, pl.ds(s * SLC, SLC)],
    )


BR = 1000


def _prep_body(x_ref, odp_ref, o_ref):
    od = odp_ref[0] + odp_ref[1]
    o_ref[...] = x_ref[...] * lax.rsqrt(jnp.maximum(od, 1.0))


_prep = pl.pallas_call(
    _prep_body,
    grid=(N // BR,),
    in_specs=[
        pl.BlockSpec((BR, D), lambda i: (i, 0)),
        pl.BlockSpec((2, BR, 1), lambda i: (0, i, 0)),
    ],
    out_specs=pl.BlockSpec((BR, D), lambda i: (i, 0)),
    out_shape=jax.ShapeDtypeStruct((N, D), jnp.float32),
)


def _make_dense(with_out_scale):
    def body(aggp_ref, idp_ref, odp_ref, w_ref, b_ref, o_ref):
        agg = aggp_ref[0] + aggp_ref[1]
        ideg = idp_ref[0] + idp_ref[1]
        x = agg * lax.rsqrt(jnp.maximum(ideg, 1.0))
        y = jnp.dot(x, w_ref[...], preferred_element_type=jnp.float32) + b_ref[...]
        y = jnp.maximum(y, 0.0)
        if with_out_scale:
            od = odp_ref[0] + odp_ref[1]
            y = y * lax.rsqrt(jnp.maximum(od, 1.0))
        o_ref[...] = y

    return pl.pallas_call(
        body,
        grid=(N // BR,),
        in_specs=[
            pl.BlockSpec((2, BR, D), lambda i: (0, i, 0)),
            pl.BlockSpec((2, BR, 1), lambda i: (0, i, 0)),
            pl.BlockSpec((2, BR, 1), lambda i: (0, i, 0)),
            pl.BlockSpec((D, D), lambda i: (0, 0)),
            pl.BlockSpec((1, D), lambda i: (0, 0)),
        ],
        out_specs=pl.BlockSpec((BR, D), lambda i: (i, 0)),
        out_shape=jax.ShapeDtypeStruct((N, D), jnp.float32),
    )


_dense_scaled = _make_dense(True)
_dense_plain = _make_dense(False)


@jax.jit
def kernel(inputs, edge_index, W1, b1, W2, b2):
    src = edge_index[0].astype(jnp.int32)
    dst = edge_index[1].astype(jnp.int32)
    src_a = src.reshape(E // CH, CH)
    dst_a = dst.reshape(E // CH, CH)
    zeros_deg = jnp.zeros((SLC,), jnp.float32)
    zeros_agg = jnp.zeros((SLC, D), jnp.float32)

    deg = _deg_kernel(src_a, dst_a, zeros_deg).reshape(NC, 2, PADN)
    odp = deg[:, 0, :N].reshape(NC, N, 1)
    idp = deg[:, 1, :N].reshape(NC, N, 1)
    b1r = b1.reshape(1, D)
    b2r = b2.reshape(1, D)

    h1n = _prep(inputs, odp)
    agg1 = _agg_kernel(h1n, src_a, dst_a, zeros_agg)
    h2n = _dense_scaled(agg1, idp, odp, W1, b1r)
    agg2 = _agg_kernel(h2n, src_a, dst_a, zeros_agg)
    return _dense_plain(agg2, idp, odp, W2, b2r)


# speedup vs baseline: 1.2743x; 1.0109x over previous
import functools

import jax
import jax.numpy as jnp
from jax import lax
from jax.experimental import pallas as pl
from jax.experimental.pallas import tpu as pltpu
from jax.experimental.pallas import tpu_sc as plsc

N = 10000
E = 320000
D = 128

NC = 2
NS = 16
NW = NC * NS

PADN = 10240
SLC = PADN // NS

CHD = 125
CPTD = E // (NW * CHD)

CH = 125
CPT = E // (NW * CH)
SB = 16
NSB = CPT // SB
NBUF = 2

_mesh = plsc.VectorSubcoreMesh(
    core_axis_name="c", subcore_axis_name="s", num_cores=NC, num_subcores=NS
)


@functools.partial(
    pl.kernel,
    out_type=jax.ShapeDtypeStruct((NC * 2 * PADN,), jnp.float32),
    mesh=_mesh,
    scratch_types=[
        pltpu.VMEM_SHARED((PADN,), jnp.float32),
        pltpu.VMEM_SHARED((PADN,), jnp.float32),
        pltpu.VMEM((CPTD, CHD), jnp.int32),
        pltpu.VMEM((CPTD, CHD), jnp.int32),
        pltpu.VMEM((128,), jnp.float32),
        pltpu.SemaphoreType.DMA,
    ],
)
def _deg_kernel(src_hbm, dst_hbm, zeros_hbm, out_hbm, od_sh, id_sh, si_v, di_v, ones_v, dsem):
    c = lax.axis_index("c")
    s = lax.axis_index("s")
    w = c * NS + s
    pltpu.sync_copy(zeros_hbm, od_sh.at[pl.ds(s * SLC, SLC)])
    pltpu.sync_copy(zeros_hbm, id_sh.at[pl.ds(s * SLC, SLC)])
    for i in range(8):
        ones_v[pl.ds(i * 16, 16)] = jnp.full((16,), 1.0, jnp.float32)
    pltpu.sync_copy(src_hbm.at[pl.ds(w * CPTD, CPTD)], si_v)
    pltpu.sync_copy(dst_hbm.at[pl.ds(w * CPTD, CPTD)], di_v)
    plsc.subcore_barrier()

    def body(j, carry):
        pltpu.async_copy(ones_v.at[pl.ds(0, CHD)], od_sh.at[si_v.at[j]], dsem, add=True)
        pltpu.async_copy(ones_v.at[pl.ds(0, CHD)], id_sh.at[di_v.at[j]], dsem, add=True)
        return carry

    lax.fori_loop(0, CPTD, body, 0)

    def drain(j, carry):
        pltpu.make_async_copy(ones_v.at[pl.ds(0, CHD)], od_sh.at[si_v.at[j]], dsem).wait()
        pltpu.make_async_copy(ones_v.at[pl.ds(0, CHD)], id_sh.at[di_v.at[j]], dsem).wait()
        return carry

    lax.fori_loop(0, CPTD, drain, 0)
    plsc.subcore_barrier()
    base = c * 2 * PADN
    pltpu.sync_copy(od_sh.at[pl.ds(s * SLC, SLC)], out_hbm.at[pl.ds(base + s * SLC, SLC)])
    pltpu.sync_copy(
        id_sh.at[pl.ds(s * SLC, SLC)], out_hbm.at[pl.ds(base + PADN + s * SLC, SLC)]
    )


@functools.partial(
    pl.kernel,
    out_type=jax.ShapeDtypeStruct((NC, PADN, D), jnp.float32),
    mesh=_mesh,
    scratch_types=[
        pltpu.VMEM_SHARED((PADN, D), jnp.float32),
        pltpu.VMEM((3, SB, CH), jnp.int32),
        pltpu.VMEM((3, SB, CH), jnp.int32),
        pltpu.VMEM((NBUF, CH, D), jnp.float32),
        pltpu.SemaphoreType.DMA,
        pltpu.SemaphoreType.DMA,
        pltpu.SemaphoreType.DMA,
        pltpu.SemaphoreType.DMA,
        pltpu.SemaphoreType.DMA,
    ],
)
def _agg_kernel(h_hbm, src_hbm, dst_hbm, zeros_hbm, out_hbm, agg_sh, si_v, di_v, rows_v, *sems):
    isem = sems[:3]
    gsem = sems[3:]
    c = lax.axis_index("c")
    s = lax.axis_index("s")
    w = c * NS + s
    row0 = w * CPT

    def pref(sb):
        pltpu.async_copy(src_hbm.at[pl.ds(row0 + sb * SB, SB)], si_v.at[sb % 3], isem[sb % 3])
        pltpu.async_copy(dst_hbm.at[pl.ds(row0 + sb * SB, SB)], di_v.at[sb % 3], isem[sb % 3])

    def wait_pref(sb):
        pltpu.make_async_copy(
            src_hbm.at[pl.ds(row0 + sb * SB, SB)], si_v.at[sb % 3], isem[sb % 3]
        ).wait()
        pltpu.make_async_copy(
            dst_hbm.at[pl.ds(row0 + sb * SB, SB)], di_v.at[sb % 3], isem[sb % 3]
        ).wait()

    pref(0)
    pltpu.sync_copy(zeros_hbm, agg_sh.at[pl.ds(s * SLC, SLC)])

    wait_pref(0)
    pref(1)
    pltpu.async_copy(h_hbm.at[si_v.at[0, 0]], rows_v.at[0], gsem[0])
    pltpu.async_copy(h_hbm.at[si_v.at[0, 1]], rows_v.at[1], gsem[1])
    plsc.subcore_barrier()
    for g in range(CPT):
        b = g % 2
        sb, jj = divmod(g, SB)
        pltpu.make_async_copy(h_hbm.at[si_v.at[sb % 3, jj]], rows_v.at[b], gsem[b]).wait()
        pltpu.sync_copy(rows_v.at[b], agg_sh.at[di_v.at[sb % 3, jj]], add=True)
        g2 = g + 2
        if g2 < CPT:
            sb2, jj2 = divmod(g2, SB)
            if jj2 == 0:
                wait_pref(sb2)
                if sb2 + 1 < NSB:
                    pref(sb2 + 1)
            pltpu.async_copy(h_hbm.at[si_v.at[sb2 % 3, jj2]], rows_v.at[b], gsem[b])
    plsc.subcore_barrier()
    pltpu.sync_copy(
        agg_sh.at[pl.ds(s * SLC, SLC)],
        out_hbm.at[c, pl.ds(s * SLC, SLC)],
    )


BR = N


def _prep_body(x_ref, odp_ref, o_ref):
    od = odp_ref[0] + odp_ref[1]
    o_ref[...] = x_ref[...] * lax.rsqrt(jnp.maximum(od, 1.0))


_prep = pl.pallas_call(
    _prep_body,
    grid=(N // BR,),
    in_specs=[
        pl.BlockSpec((BR, D), lambda i: (i, 0)),
        pl.BlockSpec((2, BR, 1), lambda i: (0, i, 0)),
    ],
    out_specs=pl.BlockSpec((BR, D), lambda i: (i, 0)),
    out_shape=jax.ShapeDtypeStruct((N, D), jnp.float32),
)


def _make_dense(with_out_scale):
    def body(aggp_ref, idp_ref, odp_ref, w_ref, b_ref, o_ref):
        agg = aggp_ref[0] + aggp_ref[1]
        ideg = idp_ref[0] + idp_ref[1]
        x = agg * lax.rsqrt(jnp.maximum(ideg, 1.0))
        y = jnp.dot(x, w_ref[...], preferred_element_type=jnp.float32) + b_ref[...]
        y = jnp.maximum(y, 0.0)
        if with_out_scale:
            od = odp_ref[0] + odp_ref[1]
            y = y * lax.rsqrt(jnp.maximum(od, 1.0))
        o_ref[...] = y

    return pl.pallas_call(
        body,
        grid=(N // BR,),
        in_specs=[
            pl.BlockSpec((2, BR, D), lambda i: (0, i, 0)),
            pl.BlockSpec((2, BR, 1), lambda i: (0, i, 0)),
            pl.BlockSpec((2, BR, 1), lambda i: (0, i, 0)),
            pl.BlockSpec((D, D), lambda i: (0, 0)),
            pl.BlockSpec((1, D), lambda i: (0, 0)),
        ],
        out_specs=pl.BlockSpec((BR, D), lambda i: (i, 0)),
        out_shape=jax.ShapeDtypeStruct((N, D), jnp.float32),
    )


_dense_scaled = _make_dense(True)
_dense_plain = _make_dense(False)


@jax.jit
def kernel(inputs, edge_index, W1, b1, W2, b2):
    src = edge_index[0].astype(jnp.int32)
    dst = edge_index[1].astype(jnp.int32)
    src_a = src.reshape(E // CH, CH)
    dst_a = dst.reshape(E // CH, CH)
    zeros_deg = jnp.zeros((SLC,), jnp.float32)
    zeros_agg = jnp.zeros((SLC, D), jnp.float32)

    deg = _deg_kernel(src_a, dst_a, zeros_deg).reshape(NC, 2, PADN)
    odp = deg[:, 0, :N].reshape(NC, N, 1)
    idp = deg[:, 1, :N].reshape(NC, N, 1)
    b1r = b1.reshape(1, D)
    b2r = b2.reshape(1, D)

    h1n = _prep(inputs, odp)
    agg1 = _agg_kernel(h1n, src_a, dst_a, zeros_agg)
    h2n = _dense_scaled(agg1, idp, odp, W1, b1r)
    agg2 = _agg_kernel(h2n, src_a, dst_a, zeros_agg)
    return _dense_plain(agg2, idp, odp, W2, b2r)


# speedup vs baseline: 1.3704x; 1.0754x over previous
import functools

import jax
import jax.numpy as jnp
from jax import lax
from jax.experimental import pallas as pl
from jax.experimental.pallas import tpu as pltpu
from jax.experimental.pallas import tpu_sc as plsc

N = 10000
E = 320000
D = 128

NC = 2
NS = 16
NW = NC * NS

PADN = 10240
SLC = PADN // NS

CHD = 125
CPTD = E // (NW * CHD)

CH = 50
CPT = E // (NW * CH)
SB = 8
NSB = CPT // SB
NBUF = 4

_mesh = plsc.VectorSubcoreMesh(
    core_axis_name="c", subcore_axis_name="s", num_cores=NC, num_subcores=NS
)


@functools.partial(
    pl.kernel,
    out_type=jax.ShapeDtypeStruct((NC * 2 * PADN,), jnp.float32),
    mesh=_mesh,
    scratch_types=[
        pltpu.VMEM_SHARED((PADN,), jnp.float32),
        pltpu.VMEM_SHARED((PADN,), jnp.float32),
        pltpu.VMEM((CPTD, CHD), jnp.int32),
        pltpu.VMEM((CPTD, CHD), jnp.int32),
        pltpu.VMEM((128,), jnp.float32),
        pltpu.SemaphoreType.DMA,
    ],
)
def _deg_kernel(src_hbm, dst_hbm, zeros_hbm, out_hbm, od_sh, id_sh, si_v, di_v, ones_v, dsem):
    c = lax.axis_index("c")
    s = lax.axis_index("s")
    w = c * NS + s
    pltpu.sync_copy(zeros_hbm, od_sh.at[pl.ds(s * SLC, SLC)])
    pltpu.sync_copy(zeros_hbm, id_sh.at[pl.ds(s * SLC, SLC)])
    for i in range(8):
        ones_v[pl.ds(i * 16, 16)] = jnp.full((16,), 1.0, jnp.float32)
    pltpu.sync_copy(src_hbm.at[pl.ds(w * CPTD, CPTD)], si_v)
    pltpu.sync_copy(dst_hbm.at[pl.ds(w * CPTD, CPTD)], di_v)
    plsc.subcore_barrier()

    def body(j, carry):
        pltpu.async_copy(ones_v.at[pl.ds(0, CHD)], od_sh.at[si_v.at[j]], dsem, add=True)
        pltpu.async_copy(ones_v.at[pl.ds(0, CHD)], id_sh.at[di_v.at[j]], dsem, add=True)
        return carry

    lax.fori_loop(0, CPTD, body, 0)

    def drain(j, carry):
        pltpu.make_async_copy(ones_v.at[pl.ds(0, CHD)], od_sh.at[si_v.at[j]], dsem).wait()
        pltpu.make_async_copy(ones_v.at[pl.ds(0, CHD)], id_sh.at[di_v.at[j]], dsem).wait()
        return carry

    lax.fori_loop(0, CPTD, drain, 0)
    plsc.subcore_barrier()
    base = c * 2 * PADN
    pltpu.sync_copy(od_sh.at[pl.ds(s * SLC, SLC)], out_hbm.at[pl.ds(base + s * SLC, SLC)])
    pltpu.sync_copy(
        id_sh.at[pl.ds(s * SLC, SLC)], out_hbm.at[pl.ds(base + PADN + s * SLC, SLC)]
    )


@functools.partial(
    pl.kernel,
    out_type=jax.ShapeDtypeStruct((NC, PADN, D), jnp.float32),
    mesh=_mesh,
    scratch_types=[
        pltpu.VMEM_SHARED((PADN, D), jnp.float32),
        pltpu.VMEM((3, SB, CH), jnp.int32),
        pltpu.VMEM((3, SB, CH), jnp.int32),
        pltpu.VMEM((NBUF, CH, D), jnp.float32),
        pltpu.SemaphoreType.DMA,
        pltpu.SemaphoreType.DMA,
        pltpu.SemaphoreType.DMA,
        pltpu.SemaphoreType.DMA,
        pltpu.SemaphoreType.DMA,
        pltpu.SemaphoreType.DMA,
        pltpu.SemaphoreType.DMA,
    ],
)
def _agg_kernel(h_hbm, src_hbm, dst_hbm, zeros_hbm, out_hbm, agg_sh, si_v, di_v, rows_v, *sems):
    isem = sems[:3]
    gsem = sems[3:]
    c = lax.axis_index("c")
    s = lax.axis_index("s")
    w = c * NS + s
    row0 = w * CPT

    def pref(sb):
        pltpu.async_copy(src_hbm.at[pl.ds(row0 + sb * SB, SB)], si_v.at[sb % 3], isem[sb % 3])
        pltpu.async_copy(dst_hbm.at[pl.ds(row0 + sb * SB, SB)], di_v.at[sb % 3], isem[sb % 3])

    def wait_pref(sb):
        pltpu.make_async_copy(
            src_hbm.at[pl.ds(row0 + sb * SB, SB)], si_v.at[sb % 3], isem[sb % 3]
        ).wait()
        pltpu.make_async_copy(
            dst_hbm.at[pl.ds(row0 + sb * SB, SB)], di_v.at[sb % 3], isem[sb % 3]
        ).wait()

    pref(0)
    pltpu.sync_copy(zeros_hbm, agg_sh.at[pl.ds(s * SLC, SLC)])

    wait_pref(0)
    pref(1)
    for b in range(NBUF):
        pltpu.async_copy(h_hbm.at[si_v.at[0, b]], rows_v.at[b], gsem[b])
    plsc.subcore_barrier()
    for g in range(CPT):
        b = g % NBUF
        sb, jj = divmod(g, SB)
        pltpu.make_async_copy(h_hbm.at[si_v.at[sb % 3, jj]], rows_v.at[b], gsem[b]).wait()
        pltpu.sync_copy(rows_v.at[b], agg_sh.at[di_v.at[sb % 3, jj]], add=True)
        g2 = g + NBUF
        if g2 < CPT:
            sb2, jj2 = divmod(g2, SB)
            if jj2 == 0:
                wait_pref(sb2)
                if sb2 + 1 < NSB:
                    pref(sb2 + 1)
            pltpu.async_copy(h_hbm.at[si_v.at[sb2 % 3, jj2]], rows_v.at[b], gsem[b])
    plsc.subcore_barrier()
    pltpu.sync_copy(
        agg_sh.at[pl.ds(s * SLC, SLC)],
        out_hbm.at[c, pl.ds(s * SLC, SLC)],
    )


BR = N


def _prep_body(x_ref, odp_ref, o_ref):
    od = odp_ref[0] + odp_ref[1]
    o_ref[...] = x_ref[...] * lax.rsqrt(jnp.maximum(od, 1.0))


_prep = pl.pallas_call(
    _prep_body,
    grid=(N // BR,),
    in_specs=[
        pl.BlockSpec((BR, D), lambda i: (i, 0)),
        pl.BlockSpec((2, BR, 1), lambda i: (0, i, 0)),
    ],
    out_specs=pl.BlockSpec((BR, D), lambda i: (i, 0)),
    out_shape=jax.ShapeDtypeStruct((N, D), jnp.float32),
)


def _make_dense(with_out_scale):
    def body(aggp_ref, idp_ref, odp_ref, w_ref, b_ref, o_ref):
        agg = aggp_ref[0] + aggp_ref[1]
        ideg = idp_ref[0] + idp_ref[1]
        x = agg * lax.rsqrt(jnp.maximum(ideg, 1.0))
        y = jnp.dot(x, w_ref[...], preferred_element_type=jnp.float32) + b_ref[...]
        y = jnp.maximum(y, 0.0)
        if with_out_scale:
            od = odp_ref[0] + odp_ref[1]
            y = y * lax.rsqrt(jnp.maximum(od, 1.0))
        o_ref[...] = y

    return pl.pallas_call(
        body,
        grid=(N // BR,),
        in_specs=[
            pl.BlockSpec((2, BR, D), lambda i: (0, i, 0)),
            pl.BlockSpec((2, BR, 1), lambda i: (0, i, 0)),
            pl.BlockSpec((2, BR, 1), lambda i: (0, i, 0)),
            pl.BlockSpec((D, D), lambda i: (0, 0)),
            pl.BlockSpec((1, D), lambda i: (0, 0)),
        ],
        out_specs=pl.BlockSpec((BR, D), lambda i: (i, 0)),
        out_shape=jax.ShapeDtypeStruct((N, D), jnp.float32),
    )


_dense_scaled = _make_dense(True)
_dense_plain = _make_dense(False)


@jax.jit
def kernel(inputs, edge_index, W1, b1, W2, b2):
    src = edge_index[0].astype(jnp.int32)
    dst = edge_index[1].astype(jnp.int32)
    src_a = src.reshape(E // CH, CH)
    dst_a = dst.reshape(E // CH, CH)
    src_d = src.reshape(E // CHD, CHD)
    dst_d = dst.reshape(E // CHD, CHD)
    zeros_deg = jnp.zeros((SLC,), jnp.float32)
    zeros_agg = jnp.zeros((SLC, D), jnp.float32)

    deg = _deg_kernel(src_d, dst_d, zeros_deg).reshape(NC, 2, PADN)
    odp = deg[:, 0, :N].reshape(NC, N, 1)
    idp = deg[:, 1, :N].reshape(NC, N, 1)
    b1r = b1.reshape(1, D)
    b2r = b2.reshape(1, D)

    h1n = _prep(inputs, odp)
    agg1 = _agg_kernel(h1n, src_a, dst_a, zeros_agg)
    h2n = _dense_scaled(agg1, idp, odp, W1, b1r)
    agg2 = _agg_kernel(h2n, src_a, dst_a, zeros_agg)
    return _dense_plain(agg2, idp, odp, W2, b2r)


# speedup vs baseline: 1.3853x; 1.0109x over previous
import functools

import jax
import jax.numpy as jnp
from jax import lax
from jax.experimental import pallas as pl
from jax.experimental.pallas import tpu as pltpu
from jax.experimental.pallas import tpu_sc as plsc

N = 10000
E = 320000
D = 128

NC = 2
NS = 16
NW = NC * NS

PADN = 10112
SLC = PADN // NS
PADN_D = 10240
SLCD = PADN_D // NS

CHD = 125
CPTD = E // (NW * CHD)

CH = 50
CPT = E // (NW * CH)
SB = 8
NSB = CPT // SB
NBUF = 6

_mesh = plsc.VectorSubcoreMesh(
    core_axis_name="c", subcore_axis_name="s", num_cores=NC, num_subcores=NS
)


@functools.partial(
    pl.kernel,
    out_type=jax.ShapeDtypeStruct((NC * 2 * PADN_D,), jnp.float32),
    mesh=_mesh,
    scratch_types=[
        pltpu.VMEM_SHARED((PADN_D,), jnp.float32),
        pltpu.VMEM_SHARED((PADN_D,), jnp.float32),
        pltpu.VMEM((CPTD, CHD), jnp.int32),
        pltpu.VMEM((CPTD, CHD), jnp.int32),
        pltpu.VMEM((128,), jnp.float32),
        pltpu.SemaphoreType.DMA,
    ],
)
def _deg_kernel(src_hbm, dst_hbm, zeros_hbm, out_hbm, od_sh, id_sh, si_v, di_v, ones_v, dsem):
    c = lax.axis_index("c")
    s = lax.axis_index("s")
    w = c * NS + s
    pltpu.sync_copy(zeros_hbm, od_sh.at[pl.ds(s * SLCD, SLCD)])
    pltpu.sync_copy(zeros_hbm, id_sh.at[pl.ds(s * SLCD, SLCD)])
    for i in range(8):
        ones_v[pl.ds(i * 16, 16)] = jnp.full((16,), 1.0, jnp.float32)
    pltpu.sync_copy(src_hbm.at[pl.ds(w * CPTD, CPTD)], si_v)
    pltpu.sync_copy(dst_hbm.at[pl.ds(w * CPTD, CPTD)], di_v)
    plsc.subcore_barrier()

    def body(j, carry):
        pltpu.async_copy(ones_v.at[pl.ds(0, CHD)], od_sh.at[si_v.at[j]], dsem, add=True)
        pltpu.async_copy(ones_v.at[pl.ds(0, CHD)], id_sh.at[di_v.at[j]], dsem, add=True)
        return carry

    lax.fori_loop(0, CPTD, body, 0)

    def drain(j, carry):
        pltpu.make_async_copy(ones_v.at[pl.ds(0, CHD)], od_sh.at[si_v.at[j]], dsem).wait()
        pltpu.make_async_copy(ones_v.at[pl.ds(0, CHD)], id_sh.at[di_v.at[j]], dsem).wait()
        return carry

    lax.fori_loop(0, CPTD, drain, 0)
    plsc.subcore_barrier()
    base = c * 2 * PADN_D
    pltpu.sync_copy(od_sh.at[pl.ds(s * SLCD, SLCD)], out_hbm.at[pl.ds(base + s * SLCD, SLCD)])
    pltpu.sync_copy(
        id_sh.at[pl.ds(s * SLCD, SLCD)], out_hbm.at[pl.ds(base + PADN_D + s * SLCD, SLCD)]
    )


@functools.partial(
    pl.kernel,
    out_type=jax.ShapeDtypeStruct((NC, PADN, D), jnp.float32),
    mesh=_mesh,
    scratch_types=[
        pltpu.VMEM_SHARED((PADN, D), jnp.float32),
        pltpu.VMEM((3, SB, CH), jnp.int32),
        pltpu.VMEM((3, SB, CH), jnp.int32),
        pltpu.VMEM((NBUF, CH, D), jnp.float32),
        pltpu.SemaphoreType.DMA,
        pltpu.SemaphoreType.DMA,
        pltpu.SemaphoreType.DMA,
        pltpu.SemaphoreType.DMA,
        pltpu.SemaphoreType.DMA,
        pltpu.SemaphoreType.DMA,
        pltpu.SemaphoreType.DMA,
        pltpu.SemaphoreType.DMA,
        pltpu.SemaphoreType.DMA,
    ],
)
def _agg_kernel(h_hbm, src_hbm, dst_hbm, zeros_hbm, out_hbm, agg_sh, si_v, di_v, rows_v, *sems):
    isem = sems[:3]
    gsem = sems[3:]
    c = lax.axis_index("c")
    s = lax.axis_index("s")
    w = c * NS + s
    row0 = w * CPT

    def pref(sb):
        pltpu.async_copy(src_hbm.at[pl.ds(row0 + sb * SB, SB)], si_v.at[sb % 3], isem[sb % 3])
        pltpu.async_copy(dst_hbm.at[pl.ds(row0 + sb * SB, SB)], di_v.at[sb % 3], isem[sb % 3])

    def wait_pref(sb):
        pltpu.make_async_copy(
            src_hbm.at[pl.ds(row0 + sb * SB, SB)], si_v.at[sb % 3], isem[sb % 3]
        ).wait()
        pltpu.make_async_copy(
            dst_hbm.at[pl.ds(row0 + sb * SB, SB)], di_v.at[sb % 3], isem[sb % 3]
        ).wait()

    pref(0)
    pltpu.sync_copy(zeros_hbm, agg_sh.at[pl.ds(s * SLC, SLC)])

    wait_pref(0)
    pref(1)
    for b in range(NBUF):
        pltpu.async_copy(h_hbm.at[si_v.at[0, b]], rows_v.at[b], gsem[b])
    plsc.subcore_barrier()
    for g in range(CPT):
        b = g % NBUF
        sb, jj = divmod(g, SB)
        pltpu.make_async_copy(h_hbm.at[si_v.at[sb % 3, jj]], rows_v.at[b], gsem[b]).wait()
        pltpu.sync_copy(rows_v.at[b], agg_sh.at[di_v.at[sb % 3, jj]], add=True)
        g2 = g + NBUF
        if g2 < CPT:
            sb2, jj2 = divmod(g2, SB)
            if jj2 == 0:
                wait_pref(sb2)
                if sb2 + 1 < NSB:
                    pref(sb2 + 1)
            pltpu.async_copy(h_hbm.at[si_v.at[sb2 % 3, jj2]], rows_v.at[b], gsem[b])
    plsc.subcore_barrier()
    pltpu.sync_copy(
        agg_sh.at[pl.ds(s * SLC, SLC)],
        out_hbm.at[c, pl.ds(s * SLC, SLC)],
    )


BR = N


def _prep_body(x_ref, odp_ref, o_ref):
    od = odp_ref[0] + odp_ref[1]
    o_ref[...] = x_ref[...] * lax.rsqrt(jnp.maximum(od, 1.0))


_prep = pl.pallas_call(
    _prep_body,
    grid=(N // BR,),
    in_specs=[
        pl.BlockSpec((BR, D), lambda i: (i, 0)),
        pl.BlockSpec((2, BR, 1), lambda i: (0, i, 0)),
    ],
    out_specs=pl.BlockSpec((BR, D), lambda i: (i, 0)),
    out_shape=jax.ShapeDtypeStruct((N, D), jnp.float32),
)


def _make_dense(with_out_scale):
    def body(aggp_ref, idp_ref, odp_ref, w_ref, b_ref, o_ref):
        agg = aggp_ref[0] + aggp_ref[1]
        ideg = idp_ref[0] + idp_ref[1]
        x = agg * lax.rsqrt(jnp.maximum(ideg, 1.0))
        y = jnp.dot(x, w_ref[...], preferred_element_type=jnp.float32) + b_ref[...]
        y = jnp.maximum(y, 0.0)
        if with_out_scale:
            od = odp_ref[0] + odp_ref[1]
            y = y * lax.rsqrt(jnp.maximum(od, 1.0))
        o_ref[...] = y

    return pl.pallas_call(
        body,
        grid=(N // BR,),
        in_specs=[
            pl.BlockSpec((2, BR, D), lambda i: (0, i, 0)),
            pl.BlockSpec((2, BR, 1), lambda i: (0, i, 0)),
            pl.BlockSpec((2, BR, 1), lambda i: (0, i, 0)),
            pl.BlockSpec((D, D), lambda i: (0, 0)),
            pl.BlockSpec((1, D), lambda i: (0, 0)),
        ],
        out_specs=pl.BlockSpec((BR, D), lambda i: (i, 0)),
        out_shape=jax.ShapeDtypeStruct((N, D), jnp.float32),
    )


_dense_scaled = _make_dense(True)
_dense_plain = _make_dense(False)


@jax.jit
def kernel(inputs, edge_index, W1, b1, W2, b2):
    src = edge_index[0].astype(jnp.int32)
    dst = edge_index[1].astype(jnp.int32)
    src_a = src.reshape(E // CH, CH)
    dst_a = dst.reshape(E // CH, CH)
    src_d = src.reshape(E // CHD, CHD)
    dst_d = dst.reshape(E // CHD, CHD)
    zeros_deg = jnp.zeros((SLCD,), jnp.float32)
    zeros_agg = jnp.zeros((SLC, D), jnp.float32)

    deg = _deg_kernel(src_d, dst_d, zeros_deg).reshape(NC, 2, PADN_D)
    odp = deg[:, 0, :N].reshape(NC, N, 1)
    idp = deg[:, 1, :N].reshape(NC, N, 1)
    b1r = b1.reshape(1, D)
    b2r = b2.reshape(1, D)

    h1n = _prep(inputs, odp)
    agg1 = _agg_kernel(h1n, src_a, dst_a, zeros_agg)
    h2n = _dense_scaled(agg1, idp, odp, W1, b1r)
    agg2 = _agg_kernel(h2n, src_a, dst_a, zeros_agg)
    return _dense_plain(agg2, idp, odp, W2, b2r)
